# baseline (speedup 1.0000x reference)
import functools

import jax
import jax.numpy as jnp
from jax import lax
from jax.experimental import pallas as pl
from jax.experimental.pallas import tpu as pltpu
from jax.experimental.pallas import tpu_sc as plsc

N_NODES = 10000
N_EDGES = 320000
D = 128
N_GRAPHS = 256
NC = 2
NS = 16
NW = NC * NS
EPW = N_EDGES // NW
ECH = 80
NCHUNK = EPW // ECH
NBUF = 4
RCH = 80
NODE_CHUNKS = N_NODES // RCH

def _sc_agg_readout_body(h_hbm, e2_hbm, gid_hbm, zeros_hbm,
                         agg_out, g_out, *scr):
    idx2 = scr[0:NBUF]
    rows = scr[NBUF:2 * NBUF]
    seml = scr[2 * NBUF:3 * NBUF]
    semg = scr[3 * NBUF:4 * NBUF]
    sems = scr[4 * NBUF:5 * NBUF]
    acc, gacc = scr[5 * NBUF:]

    cid = lax.axis_index("c")
    sid = lax.axis_index("s")
    wid = sid * NC + cid

    @pl.when(sid == 0)
    def _init():
        pltpu.sync_copy(zeros_hbm, acc)
        pltpu.sync_copy(zeros_hbm.at[pl.ds(0, N_GRAPHS)], gacc)

    plsc.subcore_barrier()

    base = wid * NCHUNK

    def start_l(c, b):
        pltpu.async_copy(e2_hbm.at[base + c], idx2[b], seml[b])

    def wait_l(b):
        pltpu.make_async_copy(e2_hbm.at[0], idx2[b], seml[b]).wait()

    def start_g(b):
        pltpu.async_copy(h_hbm.at[idx2[b].at[0]], rows[b], semg[b])

    def wait_g(b):
        pltpu.make_async_copy(h_hbm.at[idx2[b].at[0]], rows[b], semg[b]).wait()

    def start_s(b):
        pltpu.async_copy(rows[b], acc.at[idx2[b].at[1]], sems[b], add=True)

    def wait_s(b):
        pltpu.make_async_copy(rows[b], acc.at[idx2[b].at[1]], sems[b]).wait()

    start_l(0, 0)
    start_l(1, 1)
    wait_l(0)
    start_g(0)

    def estep(i, carry):
        for b in range(NBUF):
            c = NBUF * i + b
            wait_l((b + 1) % NBUF)
            start_g((b + 1) % NBUF)
            wait_g(b)
            start_s(b)
            if b >= 2:
                wait_s((b + 2) % NBUF)
            else:
                @pl.when(i > 0)
                def _():
                    wait_s((b + 2) % NBUF)
            start_l(c + 2, (b + 2) % NBUF)
        return carry

    NMAIN = (NCHUNK - 2) // NBUF
    lax.fori_loop(0, NMAIN, estep, 0, unroll=False)

    for c in range(NMAIN * NBUF, NCHUNK):
        b = c % NBUF
        if c + 1 < NCHUNK:
            wait_l((b + 1) % NBUF)
            start_g((b + 1) % NBUF)
        wait_g(b)
        start_s(b)
        if c + 2 < NCHUNK:
            wait_s((b + 2) % NBUF)
            start_l(c + 2, (b + 2) % NBUF)

    for c in range(NCHUNK - NBUF, NCHUNK):
        wait_s(c % NBUF)

    def rstep(k, carry):
        c = wid + NW * k

        @pl.when(c < NODE_CHUNKS)
        def _():
            off = c * RCH
            pltpu.sync_copy(h_hbm.at[pl.ds(off, RCH)], rows[0].at[pl.ds(0, RCH)])
            pltpu.sync_copy(gid_hbm.at[pl.ds(off, RCH)], idx2[0].at[0])
            pltpu.sync_copy(rows[0].at[pl.ds(0, RCH)], gacc.at[idx2[0].at[0]], add=True)

        return carry

    lax.fori_loop(0, (NODE_CHUNKS + NW - 1) // NW, rstep, 0)

    plsc.subcore_barrier()

    @pl.when(sid == 0)
    def _writeout():
        pltpu.sync_copy(acc, agg_out.at[cid])
        pltpu.sync_copy(gacc, g_out.at[cid])


@functools.cache
def _sc_agg_readout_kernel():
    mesh = plsc.VectorSubcoreMesh(core_axis_name="c", subcore_axis_name="s")
    scratch = (
        [pltpu.VMEM((2, ECH), jnp.int32)] * NBUF
        + [pltpu.VMEM((ECH, D), jnp.float32)] * NBUF
        + [pltpu.SemaphoreType.DMA] * (3 * NBUF)
        + [
            pltpu.VMEM_SHARED((N_NODES, D), jnp.float32),
            pltpu.VMEM_SHARED((N_GRAPHS, D), jnp.float32),
        ]
    )
    return functools.partial(
        pl.kernel,
        out_type=(
            jax.ShapeDtypeStruct((NC, N_NODES, D), jnp.float32),
            jax.ShapeDtypeStruct((NC, N_GRAPHS, D), jnp.float32),
        ),
        mesh=mesh,
        scratch_types=scratch,
    )(_sc_agg_readout_body)


def _sc_agg_readout(h, e2, gid, zeros):
    return _sc_agg_readout_kernel()(h, e2, gid, zeros)


def _sc_readout_body(h_hbm, gid_hbm, zeros_hbm, g_out,
                     idx_a, rows, gacc):
    cid = lax.axis_index("c")
    sid = lax.axis_index("s")
    wid = sid * NC + cid

    @pl.when(sid == 0)
    def _init():
        pltpu.sync_copy(zeros_hbm.at[pl.ds(0, N_GRAPHS)], gacc)

    plsc.subcore_barrier()

    def rstep(k, carry):
        c = wid + NW * k

        @pl.when(c < NODE_CHUNKS)
        def _():
            off = c * RCH
            pltpu.sync_copy(h_hbm.at[pl.ds(off, RCH)], rows)
            pltpu.sync_copy(gid_hbm.at[pl.ds(off, RCH)], idx_a)
            pltpu.sync_copy(rows, gacc.at[idx_a], add=True)

        return carry

    lax.fori_loop(0, (NODE_CHUNKS + NW - 1) // NW, rstep, 0)

    plsc.subcore_barrier()

    @pl.when(sid == 0)
    def _writeout():
        pltpu.sync_copy(gacc, g_out.at[cid])


@functools.cache
def _sc_readout_kernel():
    mesh = plsc.VectorSubcoreMesh(core_axis_name="c", subcore_axis_name="s")
    return functools.partial(
        pl.kernel,
        out_type=jax.ShapeDtypeStruct((NC, N_GRAPHS, D), jnp.float32),
        mesh=mesh,
        scratch_types=[
            pltpu.VMEM((RCH,), jnp.int32),
            pltpu.VMEM((RCH, D), jnp.float32),
            pltpu.VMEM_SHARED((N_GRAPHS, D), jnp.float32),
        ],
    )(_sc_readout_body)


def _sc_readout(h, gid, zeros):
    return _sc_readout_kernel()(h, gid, zeros)


_ROWS_BLK = 1000
_GRID = N_NODES // _ROWS_BLK


def _proj_body(x_ref, w_ref, b_ref, o_ref):
    o_ref[...] = (
        jnp.dot(x_ref[...], w_ref[...], preferred_element_type=jnp.float32)
        + b_ref[...]
    )


def _proj(x, W, b):
    return pl.pallas_call(
        _proj_body,
        grid=(_GRID,),
        in_specs=[
            pl.BlockSpec((_ROWS_BLK, D), lambda i: (i, 0)),
            pl.BlockSpec((D, D), lambda i: (0, 0)),
            pl.BlockSpec((1, D), lambda i: (0, 0)),
        ],
        out_specs=pl.BlockSpec((_ROWS_BLK, D), lambda i: (i, 0)),
        out_shape=jax.ShapeDtypeStruct((N_NODES, D), jnp.float32),
    )(x, W, b)


def _gin_body(h_ref, a0_ref, a1_ref, w_ref, b_ref, s_ref, o_ref):
    t = s_ref[...] * h_ref[...] + a0_ref[...] + a1_ref[...]
    o_ref[...] = jnp.maximum(
        jnp.dot(t, w_ref[...], preferred_element_type=jnp.float32) + b_ref[...],
        0.0,
    )


def _gin_combine(h, a0, a1, W, b, scale):
    return pl.pallas_call(
        _gin_body,
        grid=(_GRID,),
        in_specs=[
            pl.BlockSpec((_ROWS_BLK, D), lambda i: (i, 0)),
            pl.BlockSpec((_ROWS_BLK, D), lambda i: (i, 0)),
            pl.BlockSpec((_ROWS_BLK, D), lambda i: (i, 0)),
            pl.BlockSpec((D, D), lambda i: (0, 0)),
            pl.BlockSpec((1, D), lambda i: (0, 0)),
            pl.BlockSpec((1, D), lambda i: (0, 0)),
        ],
        out_specs=pl.BlockSpec((_ROWS_BLK, D), lambda i: (i, 0)),
        out_shape=jax.ShapeDtypeStruct((N_NODES, D), jnp.float32),
    )(h, a0, a1, W, b, scale)


def _head_body(g_ref, a_ref, w1g_ref, w1a_ref, b1_ref, w2_ref, b2_ref,
               wo_ref, bo_ref, o_ref):
    acc = (
        jnp.dot(a_ref[...], w1a_ref[...], preferred_element_type=jnp.float32)
        + b1_ref[...]
    )
    for i in range(4):
        gi = g_ref[i, 0] + g_ref[i, 1]
        acc = acc + jnp.dot(
            gi, w1g_ref[i * D:(i + 1) * D, :], preferred_element_type=jnp.float32
        )
    z = jnp.maximum(acc, 0.0)
    z = jnp.maximum(
        jnp.dot(z, w2_ref[...], preferred_element_type=jnp.float32) + b2_ref[...],
        0.0,
    )
    o_ref[...] = (
        jnp.dot(z, wo_ref[...], preferred_element_type=jnp.float32) + bo_ref[...]
    )


def _head(G, A, W1g, W1a, b1, W2, b2, Wo, bo):
    return pl.pallas_call(
        _head_body,
        out_shape=jax.ShapeDtypeStruct((N_GRAPHS, D), jnp.float32),
    )(G, A, W1g, W1a, b1, W2, b2, Wo, bo)


def kernel(x, edge_index, graph_ids, x_adduct, W_proj, b_proj, W_gin, b_gin,
           eps, W1, b1, W2, b2, Wo, bo):
    e2 = edge_index.reshape(2, NW, NCHUNK, ECH).transpose(1, 2, 0, 3)
    e2 = e2.reshape(NW * NCHUNK, 2, ECH)
    zeros = jnp.zeros((N_NODES, D), jnp.float32)

    h = _proj(x, W_proj, b_proj.reshape(1, D))
    g_parts = []
    for i in range(3):
        agg_p, g_p = _sc_agg_readout(h, e2, graph_ids, zeros)
        g_parts.append(g_p)
        scale = jnp.full((1, D), 1.0 + eps[i], jnp.float32)
        h = _gin_combine(h, agg_p[0], agg_p[1], W_gin[i],
                         b_gin[i].reshape(1, D), scale)
    g_parts.append(_sc_readout(h, graph_ids, zeros))

    G = jnp.stack(g_parts)
    A = jnp.pad(x_adduct, ((0, 0), (0, D - 8)))
    W1g = W1[: 4 * D]
    W1a = jnp.pad(W1[4 * D:], ((0, D - 8), (0, 0)))
    Wop = jnp.pad(Wo, ((0, 0), (0, D - 1)))
    bop = jnp.pad(bo, (0, D - 1)).reshape(1, D)

    out = _head(G, A, W1g, W1a, b1.reshape(1, -1), W2, b2.reshape(1, -1),
                Wop, bop)
    return out[:, :1]


# speedup vs baseline: 1.0074x; 1.0074x over previous
import functools

import jax
import jax.numpy as jnp
from jax import lax
from jax.experimental import pallas as pl
from jax.experimental.pallas import tpu as pltpu
from jax.experimental.pallas import tpu_sc as plsc

N_NODES = 10000
N_EDGES = 320000
D = 128
N_GRAPHS = 256
DENSE = 256
NC = 2
NS = 16
NW = NC * NS
EPW = N_EDGES // NW
ECH = 80
NCHUNK = EPW // ECH
NBUF = 4
RCH = 80
NODE_CHUNKS = N_NODES // RCH

def _sc_agg_body(h_hbm, e2_hbm, zeros_hbm, agg_out, *scr):
    idx2 = scr[0:NBUF]
    rows = scr[NBUF:2 * NBUF]
    seml = scr[2 * NBUF:3 * NBUF]
    semg = scr[3 * NBUF:4 * NBUF]
    sems = scr[4 * NBUF:5 * NBUF]
    acc = scr[5 * NBUF]

    cid = lax.axis_index("c")
    sid = lax.axis_index("s")
    wid = sid * NC + cid

    base = wid * NCHUNK

    def start_l(c, b):
        pltpu.async_copy(e2_hbm.at[base + c], idx2[b], seml[b])

    def wait_l(b):
        pltpu.make_async_copy(e2_hbm.at[0], idx2[b], seml[b]).wait()

    def start_g(b):
        pltpu.async_copy(h_hbm.at[idx2[b].at[0]], rows[b], semg[b])

    def wait_g(b):
        pltpu.make_async_copy(h_hbm.at[idx2[b].at[0]], rows[b], semg[b]).wait()

    def start_s(b):
        pltpu.async_copy(rows[b], acc.at[idx2[b].at[1]], sems[b], add=True)

    def wait_s(b):
        pltpu.make_async_copy(rows[b], acc.at[idx2[b].at[1]], sems[b]).wait()

    start_l(0, 0)
    start_l(1, 1)

    rpt = 624
    pltpu.sync_copy(zeros_hbm.at[pl.ds(sid * rpt, rpt)],
                    acc.at[pl.ds(sid * rpt, rpt)])

    @pl.when(sid == 0)
    def _init_tail():
        pltpu.sync_copy(zeros_hbm.at[pl.ds(NS * rpt, N_NODES - NS * rpt)],
                        acc.at[pl.ds(NS * rpt, N_NODES - NS * rpt)])

    plsc.subcore_barrier()

    wait_l(0)
    start_g(0)

    def estep(i, carry):
        for b in range(NBUF):
            c = NBUF * i + b
            wait_l((b + 1) % NBUF)
            start_g((b + 1) % NBUF)
            wait_g(b)
            start_s(b)
            if b >= 2:
                wait_s((b + 2) % NBUF)
            else:
                @pl.when(i > 0)
                def _():
                    wait_s((b + 2) % NBUF)
            start_l(c + 2, (b + 2) % NBUF)
        return carry

    NMAIN = (NCHUNK - 2) // NBUF
    lax.fori_loop(0, NMAIN, estep, 0, unroll=False)

    for c in range(NMAIN * NBUF, NCHUNK):
        b = c % NBUF
        if c + 1 < NCHUNK:
            wait_l((b + 1) % NBUF)
            start_g((b + 1) % NBUF)
        wait_g(b)
        start_s(b)
        if c + 2 < NCHUNK:
            wait_s((b + 2) % NBUF)
            start_l(c + 2, (b + 2) % NBUF)

    for c in range(NCHUNK - NBUF, NCHUNK):
        wait_s(c % NBUF)

    plsc.subcore_barrier()

    pltpu.sync_copy(acc.at[pl.ds(sid * rpt, rpt)],
                    agg_out.at[cid, pl.ds(sid * rpt, rpt)])

    @pl.when(sid == 0)
    def _write_tail():
        pltpu.sync_copy(acc.at[pl.ds(NS * rpt, N_NODES - NS * rpt)],
                        agg_out.at[cid, pl.ds(NS * rpt, N_NODES - NS * rpt)])


@functools.cache
def _sc_agg_kernel():
    mesh = plsc.VectorSubcoreMesh(core_axis_name="c", subcore_axis_name="s")
    scratch = (
        [pltpu.VMEM((2, ECH), jnp.int32)] * NBUF
        + [pltpu.VMEM((ECH, D), jnp.float32)] * NBUF
        + [pltpu.SemaphoreType.DMA] * (3 * NBUF)
        + [pltpu.VMEM_SHARED((N_NODES, D), jnp.float32)]
    )
    return functools.partial(
        pl.kernel,
        out_type=jax.ShapeDtypeStruct((NC, N_NODES, D), jnp.float32),
        mesh=mesh,
        scratch_types=scratch,
    )(_sc_agg_body)


def _sc_agg(h, e2, zeros):
    return _sc_agg_kernel()(h, e2, zeros)


_ROWS_BLK = 1000
_GRID = N_NODES // _ROWS_BLK


def _proj_body(x_ref, w_ref, b_ref, o_ref):
    o_ref[...] = (
        jnp.dot(x_ref[...], w_ref[...], preferred_element_type=jnp.float32)
        + b_ref[...]
    )


def _proj(x, W, b):
    return pl.pallas_call(
        _proj_body,
        grid=(_GRID,),
        in_specs=[
            pl.BlockSpec((_ROWS_BLK, D), lambda i: (i, 0)),
            pl.BlockSpec((D, D), lambda i: (0, 0)),
            pl.BlockSpec((1, D), lambda i: (0, 0)),
        ],
        out_specs=pl.BlockSpec((_ROWS_BLK, D), lambda i: (i, 0)),
        out_shape=jax.ShapeDtypeStruct((N_NODES, D), jnp.float32),
    )(x, W, b)


def _gin_body(h_ref, a0_ref, a1_ref, w_ref, b_ref, s_ref, o_ref):
    t = s_ref[...] * h_ref[...] + a0_ref[...] + a1_ref[...]
    o_ref[...] = jnp.maximum(
        jnp.dot(t, w_ref[...], preferred_element_type=jnp.float32) + b_ref[...],
        0.0,
    )


def _gin_combine(h, a0, a1, W, b, scale):
    return pl.pallas_call(
        _gin_body,
        grid=(_GRID,),
        in_specs=[
            pl.BlockSpec((_ROWS_BLK, D), lambda i: (i, 0)),
            pl.BlockSpec((_ROWS_BLK, D), lambda i: (i, 0)),
            pl.BlockSpec((_ROWS_BLK, D), lambda i: (i, 0)),
            pl.BlockSpec((D, D), lambda i: (0, 0)),
            pl.BlockSpec((1, D), lambda i: (0, 0)),
            pl.BlockSpec((1, D), lambda i: (0, 0)),
        ],
        out_specs=pl.BlockSpec((_ROWS_BLK, D), lambda i: (i, 0)),
        out_shape=jax.ShapeDtypeStruct((N_NODES, D), jnp.float32),
    )(h, a0, a1, W, b, scale)


def _rhead_body(h0_ref, h1_ref, h2_ref, h3_ref, gid_ref, a_ref,
                w1g_ref, w1a_ref, b1_ref, w2_ref, b2_ref, wo_ref, bo_ref,
                o_ref, g0, g1, g2, g3):
    i = pl.program_id(0)
    gs = (g0, g1, g2, g3)

    @pl.when(i == 0)
    def _zero():
        for g in gs:
            g[...] = jnp.zeros((N_GRAPHS, D), jnp.float32)

    iota = jax.lax.broadcasted_iota(jnp.int32, (_ROWS_BLK, N_GRAPHS), 1)
    m = (gid_ref[...] == iota).astype(jnp.float32)
    cn = (((0,), (0,)), ((), ()))
    for g, h_ref in zip(gs, (h0_ref, h1_ref, h2_ref, h3_ref)):
        g[...] += jax.lax.dot_general(m, h_ref[...], cn,
                                      preferred_element_type=jnp.float32,
                                      precision=jax.lax.Precision.HIGHEST)

    @pl.when(i == _GRID - 1)
    def _head():
        acc = (
            jnp.dot(a_ref[...], w1a_ref[...], preferred_element_type=jnp.float32)
            + b1_ref[...]
        )
        for l, g in enumerate(gs):
            acc = acc + jnp.dot(
                g[...], w1g_ref[l * D:(l + 1) * D, :],
                preferred_element_type=jnp.float32,
            )
        z = jnp.maximum(acc, 0.0)
        z = jnp.maximum(
            jnp.dot(z, w2_ref[...], preferred_element_type=jnp.float32)
            + b2_ref[...],
            0.0,
        )
        o_ref[...] = (
            jnp.dot(z, wo_ref[...], preferred_element_type=jnp.float32)
            + bo_ref[...]
        )


def _readout_head(h0, h1, h2, h3, gid2, A, W1g, W1a, b1, W2, b2, Wo, bo):
    hspec = pl.BlockSpec((_ROWS_BLK, D), lambda i: (i, 0))
    full = lambda shape: pl.BlockSpec(shape, lambda i: (0,) * len(shape))
    return pl.pallas_call(
        _rhead_body,
        grid=(_GRID,),
        in_specs=[
            hspec, hspec, hspec, hspec,
            pl.BlockSpec((_ROWS_BLK, 1), lambda i: (i, 0)),
            full((N_GRAPHS, D)),
            full((4 * D, DENSE)),
            full((D, DENSE)),
            full((1, DENSE)),
            full((DENSE, DENSE)),
            full((1, DENSE)),
            full((DENSE, D)),
            full((1, D)),
        ],
        out_specs=full((N_GRAPHS, D)),
        out_shape=jax.ShapeDtypeStruct((N_GRAPHS, D), jnp.float32),
        scratch_shapes=[pltpu.VMEM((N_GRAPHS, D), jnp.float32)] * 4,
    )(h0, h1, h2, h3, gid2, A, W1g, W1a, b1, W2, b2, Wo, bo)


def kernel(x, edge_index, graph_ids, x_adduct, W_proj, b_proj, W_gin, b_gin,
           eps, W1, b1, W2, b2, Wo, bo):
    e2 = edge_index.reshape(2, NW, NCHUNK, ECH).transpose(1, 2, 0, 3)
    e2 = e2.reshape(NW * NCHUNK, 2, ECH)
    zeros = jnp.zeros((N_NODES, D), jnp.float32)

    hs = [_proj(x, W_proj, b_proj.reshape(1, D))]
    for i in range(3):
        agg_p = _sc_agg(hs[-1], e2, zeros)
        scale = jnp.full((1, D), 1.0 + eps[i], jnp.float32)
        hs.append(_gin_combine(hs[-1], agg_p[0], agg_p[1], W_gin[i],
                               b_gin[i].reshape(1, D), scale))

    gid2 = graph_ids.reshape(N_NODES, 1)
    A = jnp.pad(x_adduct, ((0, 0), (0, D - 8)))
    W1g = W1[: 4 * D]
    W1a = jnp.pad(W1[4 * D:], ((0, D - 8), (0, 0)))
    Wop = jnp.pad(Wo, ((0, 0), (0, D - 1)))
    bop = jnp.pad(bo, (0, D - 1)).reshape(1, D)

    out = _readout_head(hs[0], hs[1], hs[2], hs[3], gid2, A, W1g, W1a,
                        b1.reshape(1, -1), W2, b2.reshape(1, -1), Wop, bop)
    return out[:, :1]


# speedup vs baseline: 1.1264x; 1.1182x over previous
import functools

import jax
import jax.numpy as jnp
from jax import lax
from jax.experimental import pallas as pl
from jax.experimental.pallas import tpu as pltpu
from jax.experimental.pallas import tpu_sc as plsc

N_NODES = 10000
N_EDGES = 320000
D = 128
N_GRAPHS = 256
DENSE = 256
NC = 2
NS = 16
NW = NC * NS
EPW = N_EDGES // NW
ECH = 128
NCHUNK = EPW // ECH
TAIL = EPW - NCHUNK * ECH
NBUF = 3
RCH = 80
NODE_CHUNKS = N_NODES // RCH

def _sc_agg_body(h_hbm, e2_hbm, et_hbm, zeros_hbm, agg_out, *scr):
    idx2 = scr[0:NBUF]
    rows = scr[NBUF:2 * NBUF]
    seml = scr[2 * NBUF:3 * NBUF]
    semg = scr[3 * NBUF:4 * NBUF]
    sems = scr[4 * NBUF:5 * NBUF]
    idx_t, acc = scr[5 * NBUF:]

    cid = lax.axis_index("c")
    sid = lax.axis_index("s")
    wid = sid * NC + cid

    base = wid * NCHUNK

    def start_l(c, b):
        pltpu.async_copy(e2_hbm.at[base + c], idx2[b], seml[b])

    def wait_l(b):
        pltpu.make_async_copy(e2_hbm.at[0], idx2[b], seml[b]).wait()

    def start_g(b):
        pltpu.async_copy(h_hbm.at[idx2[b].at[0]], rows[b], semg[b])

    def wait_g(b):
        pltpu.make_async_copy(h_hbm.at[idx2[b].at[0]], rows[b], semg[b]).wait()

    def start_s(b):
        pltpu.async_copy(rows[b], acc.at[idx2[b].at[1]], sems[b], add=True)

    def wait_s(b):
        pltpu.make_async_copy(rows[b], acc.at[idx2[b].at[1]], sems[b]).wait()

    start_l(0, 0)
    start_l(1, 1)

    rpt = 624
    pltpu.sync_copy(zeros_hbm.at[pl.ds(sid * rpt, rpt)],
                    acc.at[pl.ds(sid * rpt, rpt)])

    @pl.when(sid == 0)
    def _init_tail():
        pltpu.sync_copy(zeros_hbm.at[pl.ds(NS * rpt, N_NODES - NS * rpt)],
                        acc.at[pl.ds(NS * rpt, N_NODES - NS * rpt)])

    plsc.subcore_barrier()

    wait_l(0)
    start_g(0)

    def estep(i, carry):
        for b in range(NBUF):
            c = NBUF * i + b
            wait_l((b + 1) % NBUF)
            start_g((b + 1) % NBUF)
            wait_g(b)
            start_s(b)
            if b >= 1:
                wait_s((b + 2) % NBUF)
            else:
                @pl.when(i > 0)
                def _():
                    wait_s((b + 2) % NBUF)
            start_l(c + 2, (b + 2) % NBUF)
        return carry

    NMAIN = (NCHUNK - 2) // NBUF
    lax.fori_loop(0, NMAIN, estep, 0, unroll=False)

    for c in range(NMAIN * NBUF, NCHUNK):
        b = c % NBUF
        if c + 1 < NCHUNK:
            wait_l((b + 1) % NBUF)
            start_g((b + 1) % NBUF)
        wait_g(b)
        start_s(b)
        if c + 2 < NCHUNK:
            wait_s((b + 2) % NBUF)
            start_l(c + 2, (b + 2) % NBUF)

    for c in range(NCHUNK - NBUF, NCHUNK):
        wait_s(c % NBUF)

    pltpu.async_copy(et_hbm.at[wid], idx_t, seml[0])
    pltpu.make_async_copy(et_hbm.at[0], idx_t, seml[0]).wait()
    pltpu.async_copy(h_hbm.at[idx_t.at[0]], rows[0].at[pl.ds(0, TAIL)],
                     semg[0]).wait()
    pltpu.sync_copy(rows[0].at[pl.ds(0, TAIL)], acc.at[idx_t.at[1]], add=True)

    plsc.subcore_barrier()

    pltpu.sync_copy(acc.at[pl.ds(sid * rpt, rpt)],
                    agg_out.at[cid, pl.ds(sid * rpt, rpt)])

    @pl.when(sid == 0)
    def _write_tail():
        pltpu.sync_copy(acc.at[pl.ds(NS * rpt, N_NODES - NS * rpt)],
                        agg_out.at[cid, pl.ds(NS * rpt, N_NODES - NS * rpt)])


@functools.cache
def _sc_agg_kernel():
    mesh = plsc.VectorSubcoreMesh(core_axis_name="c", subcore_axis_name="s")
    scratch = (
        [pltpu.VMEM((2, ECH), jnp.int32)] * NBUF
        + [pltpu.VMEM((ECH, D), jnp.float32)] * NBUF
        + [pltpu.SemaphoreType.DMA] * (3 * NBUF)
        + [pltpu.VMEM((2, TAIL), jnp.int32)]
        + [pltpu.VMEM_SHARED((N_NODES, D), jnp.float32)]
    )
    return functools.partial(
        pl.kernel,
        out_type=jax.ShapeDtypeStruct((NC, N_NODES, D), jnp.float32),
        mesh=mesh,
        scratch_types=scratch,
    )(_sc_agg_body)


def _sc_agg(h, e2, et, zeros):
    return _sc_agg_kernel()(h, e2, et, zeros)


_ROWS_BLK = 1000
_GRID = N_NODES // _ROWS_BLK


def _proj_body(x_ref, w_ref, b_ref, o_ref):
    o_ref[...] = (
        jnp.dot(x_ref[...], w_ref[...], preferred_element_type=jnp.float32)
        + b_ref[...]
    )


def _proj(x, W, b):
    return pl.pallas_call(
        _proj_body,
        grid=(_GRID,),
        in_specs=[
            pl.BlockSpec((_ROWS_BLK, D), lambda i: (i, 0)),
            pl.BlockSpec((D, D), lambda i: (0, 0)),
            pl.BlockSpec((1, D), lambda i: (0, 0)),
        ],
        out_specs=pl.BlockSpec((_ROWS_BLK, D), lambda i: (i, 0)),
        out_shape=jax.ShapeDtypeStruct((N_NODES, D), jnp.float32),
    )(x, W, b)


def _gin_body(h_ref, a0_ref, a1_ref, w_ref, b_ref, s_ref, o_ref):
    t = s_ref[...] * h_ref[...] + a0_ref[...] + a1_ref[...]
    o_ref[...] = jnp.maximum(
        jnp.dot(t, w_ref[...], preferred_element_type=jnp.float32) + b_ref[...],
        0.0,
    )


def _gin_combine(h, a0, a1, W, b, scale):
    return pl.pallas_call(
        _gin_body,
        grid=(_GRID,),
        in_specs=[
            pl.BlockSpec((_ROWS_BLK, D), lambda i: (i, 0)),
            pl.BlockSpec((_ROWS_BLK, D), lambda i: (i, 0)),
            pl.BlockSpec((_ROWS_BLK, D), lambda i: (i, 0)),
            pl.BlockSpec((D, D), lambda i: (0, 0)),
            pl.BlockSpec((1, D), lambda i: (0, 0)),
            pl.BlockSpec((1, D), lambda i: (0, 0)),
        ],
        out_specs=pl.BlockSpec((_ROWS_BLK, D), lambda i: (i, 0)),
        out_shape=jax.ShapeDtypeStruct((N_NODES, D), jnp.float32),
    )(h, a0, a1, W, b, scale)


def _rhead_body(h0_ref, h1_ref, h2_ref, h3_ref, gid_ref, a_ref,
                w1g_ref, w1a_ref, b1_ref, w2_ref, b2_ref, wo_ref, bo_ref,
                o_ref, g0, g1, g2, g3):
    i = pl.program_id(0)
    gs = (g0, g1, g2, g3)

    @pl.when(i == 0)
    def _zero():
        for g in gs:
            g[...] = jnp.zeros((N_GRAPHS, D), jnp.float32)

    iota = jax.lax.broadcasted_iota(jnp.int32, (_ROWS_BLK, N_GRAPHS), 1)
    m = (gid_ref[...] == iota).astype(jnp.float32)
    cn = (((0,), (0,)), ((), ()))
    for g, h_ref in zip(gs, (h0_ref, h1_ref, h2_ref, h3_ref)):
        g[...] += jax.lax.dot_general(m, h_ref[...], cn,
                                      preferred_element_type=jnp.float32,
                                      precision=jax.lax.Precision.HIGHEST)

    @pl.when(i == _GRID - 1)
    def _head():
        acc = (
            jnp.dot(a_ref[...], w1a_ref[...], preferred_element_type=jnp.float32)
            + b1_ref[...]
        )
        for l, g in enumerate(gs):
            acc = acc + jnp.dot(
                g[...], w1g_ref[l * D:(l + 1) * D, :],
                preferred_element_type=jnp.float32,
            )
        z = jnp.maximum(acc, 0.0)
        z = jnp.maximum(
            jnp.dot(z, w2_ref[...], preferred_element_type=jnp.float32)
            + b2_ref[...],
            0.0,
        )
        o_ref[...] = (
            jnp.dot(z, wo_ref[...], preferred_element_type=jnp.float32)
            + bo_ref[...]
        )


def _readout_head(h0, h1, h2, h3, gid2, A, W1g, W1a, b1, W2, b2, Wo, bo):
    hspec = pl.BlockSpec((_ROWS_BLK, D), lambda i: (i, 0))
    full = lambda shape: pl.BlockSpec(shape, lambda i: (0,) * len(shape))
    return pl.pallas_call(
        _rhead_body,
        grid=(_GRID,),
        in_specs=[
            hspec, hspec, hspec, hspec,
            pl.BlockSpec((_ROWS_BLK, 1), lambda i: (i, 0)),
            full((N_GRAPHS, D)),
            full((4 * D, DENSE)),
            full((D, DENSE)),
            full((1, DENSE)),
            full((DENSE, DENSE)),
            full((1, DENSE)),
            full((DENSE, D)),
            full((1, D)),
        ],
        out_specs=full((N_GRAPHS, D)),
        out_shape=jax.ShapeDtypeStruct((N_GRAPHS, D), jnp.float32),
        scratch_shapes=[pltpu.VMEM((N_GRAPHS, D), jnp.float32)] * 4,
    )(h0, h1, h2, h3, gid2, A, W1g, W1a, b1, W2, b2, Wo, bo)


def kernel(x, edge_index, graph_ids, x_adduct, W_proj, b_proj, W_gin, b_gin,
           eps, W1, b1, W2, b2, Wo, bo):
    ei = edge_index.reshape(2, NW, EPW)
    e2 = ei[:, :, :NCHUNK * ECH].reshape(2, NW, NCHUNK, ECH)
    e2 = e2.transpose(1, 2, 0, 3).reshape(NW * NCHUNK, 2, ECH)
    et = ei[:, :, NCHUNK * ECH:].transpose(1, 0, 2)
    zeros = jnp.zeros((N_NODES, D), jnp.float32)

    hs = [_proj(x, W_proj, b_proj.reshape(1, D))]
    for i in range(3):
        agg_p = _sc_agg(hs[-1], e2, et, zeros)
        scale = jnp.full((1, D), 1.0 + eps[i], jnp.float32)
        hs.append(_gin_combine(hs[-1], agg_p[0], agg_p[1], W_gin[i],
                               b_gin[i].reshape(1, D), scale))

    gid2 = graph_ids.reshape(N_NODES, 1)
    A = jnp.pad(x_adduct, ((0, 0), (0, D - 8)))
    W1g = W1[: 4 * D]
    W1a = jnp.pad(W1[4 * D:], ((0, D - 8), (0, 0)))
    Wop = jnp.pad(Wo, ((0, 0), (0, D - 1)))
    bop = jnp.pad(bo, (0, D - 1)).reshape(1, D)

    out = _readout_head(hs[0], hs[1], hs[2], hs[3], gid2, A, W1g, W1a,
                        b1.reshape(1, -1), W2, b2.reshape(1, -1), Wop, bop)
    return out[:, :1]
